# baseline (speedup 1.0000x reference)
import functools
import numpy as np
import jax
import jax.numpy as jnp
from jax.experimental import pallas as pl
from jax.experimental.pallas import tpu as pltpu

_D = 768
_H = 12
_E = 64
_FF = 3072
_PRED = 512
_CS = 64

_ROT_ENC = np.random.RandomState(42).randn(_E, 16).astype(np.float32)
_ROT_DEC = np.random.RandomState(42).randn(_E, 8).astype(np.float32)


def _pos_emb(L, d):
    pos = np.arange(L, dtype=np.float32)[:, None]
    div = np.exp(np.arange(0, d, 2, dtype=np.float32) * (-np.log(10000.0) / d))
    pe = np.zeros((L, d), dtype=np.float32)
    pe[:, 0::2] = np.sin(pos * div)
    pe[:, 1::2] = np.cos(pos * div)
    return pe

_PE = {2048: _pos_emb(2048, _D), 1024: _pos_emb(1024, _D)}


def _embed_kernel(xcat_ref, w_ref, pe_ref, o_ref):
    o_ref[0] = (jnp.dot(xcat_ref[0], w_ref[...],
                        preferred_element_type=jnp.float32) + pe_ref[...])


def _embed(x, mark, ep, pe):
    Bb, L, C = x.shape
    xcat = jnp.concatenate(
        [jnp.roll(x, 1, axis=1), x, jnp.roll(x, -1, axis=1), mark], axis=-1)
    K = xcat.shape[-1]
    Kp = 32
    xcat = jnp.pad(xcat, ((0, 0), (0, 0), (0, Kp - K)))
    w = jnp.concatenate([ep['conv'][0], ep['conv'][1], ep['conv'][2],
                         ep['timeW']], axis=0)
    w = jnp.pad(w, ((0, Kp - K), (0, 0)))
    return pl.pallas_call(
        _embed_kernel,
        grid=(Bb,),
        in_specs=[
            pl.BlockSpec((1, L, Kp), lambda b: (b, 0, 0)),
            pl.BlockSpec((Kp, _D), lambda b: (0, 0)),
            pl.BlockSpec((L, _D), lambda b: (0, 0)),
        ],
        out_specs=pl.BlockSpec((1, L, _D), lambda b: (b, 0, 0)),
        out_shape=jax.ShapeDtypeStruct((Bb, L, _D), jnp.float32),
    )(xcat, w, jnp.asarray(pe))


def _bf(x):
    return x.astype(jnp.bfloat16)


def _lsh_kernel(x_ref, wq_ref, wk_ref, wv_ref, bq_ref, bk_ref, bv_ref,
                rot_ref, o_ref, sq_ref, sk_ref, sv_ref, *, L, nch):
    x = x_ref[0]
    xb = _bf(x)
    q = (jnp.dot(xb, _bf(wq_ref[0]), preferred_element_type=jnp.float32)
         + bq_ref[0])
    k = (jnp.dot(xb, _bf(wk_ref[0]), preferred_element_type=jnp.float32)
         + bk_ref[0])
    v = (jnp.dot(xb, _bf(wv_ref[0]), preferred_element_type=jnp.float32)
         + bv_ref[0])

    nb = 2 * rot_ref.shape[-1]
    wr = jnp.dot(wq_ref[0], rot_ref[...], preferred_element_type=jnp.float32)
    rotated = (jnp.dot(x, wr, preferred_element_type=jnp.float32)
               + jnp.dot(bq_ref[0], rot_ref[...],
                         preferred_element_type=jnp.float32))
    cat = jnp.concatenate([rotated, -rotated], axis=1)
    m = jnp.max(cat, axis=1, keepdims=True)
    lane = jax.lax.broadcasted_iota(jnp.int32, (L, nb), 1)
    bucket = jnp.min(jnp.where(cat == m, lane, nb), axis=1, keepdims=True)

    onehot = (bucket == lane).astype(jnp.float32)
    cum = onehot
    s = 1
    while s < L:
        cum = cum + jnp.concatenate(
            [jnp.zeros((s, nb), jnp.float32), cum[:L - s]], axis=0)
        s *= 2
    hist = cum[L - 1:L, :]
    off = hist
    s = 1
    while s < nb:
        off = off + jnp.concatenate(
            [jnp.zeros((1, s), jnp.float32), off[:, :nb - s]], axis=1)
        s *= 2
    off = off - hist
    rank = jnp.sum(onehot * cum, axis=1, keepdims=True)
    offsel = jnp.sum(onehot * off, axis=1, keepdims=True)
    dest = (offsel + rank - 1.0).astype(jnp.int32)

    cl = jax.lax.broadcasted_iota(jnp.int32, (L, _CS), 1)

    qb, kb, vb = _bf(q), _bf(k), _bf(v)

    def gather_body(c, _):
        p = (dest == (cl + c * _CS)).astype(jnp.bfloat16)
        dn = (((0,), (0,)), ((), ()))
        idx = pl.ds(pl.multiple_of(c * _CS, _CS), _CS)
        sq_ref[idx, :] = jax.lax.dot_general(
            p, qb, dn, preferred_element_type=jnp.float32)
        sk_ref[idx, :] = jax.lax.dot_general(
            p, kb, dn, preferred_element_type=jnp.float32)
        sv_ref[idx, :] = jax.lax.dot_general(
            p, vb, dn, preferred_element_type=jnp.float32)
        return 0

    jax.lax.fori_loop(0, nch, gather_body, 0)

    o_ref[0] = jnp.zeros((L, _E), jnp.float32)

    def attn_body(c, _):
        pc = jnp.where(c == 0, nch - 1, c - 1)
        ic = pl.ds(pl.multiple_of(c * _CS, _CS), _CS)
        ip = pl.ds(pl.multiple_of(pc * _CS, _CS), _CS)
        sq = _bf(sq_ref[ic, :])
        k2 = _bf(jnp.concatenate([sk_ref[ip, :], sk_ref[ic, :]], axis=0))
        v2 = _bf(jnp.concatenate([sv_ref[ip, :], sv_ref[ic, :]], axis=0))
        dots = jax.lax.dot_general(
            sq, k2, (((1,), (1,)), ((), ())),
            preferred_element_type=jnp.float32) / 8.0
        mx = jnp.max(dots, axis=1, keepdims=True)
        e = jnp.exp(dots - mx)
        a = _bf(e / jnp.sum(e, axis=1, keepdims=True))
        oc = jnp.dot(a, v2, preferred_element_type=jnp.float32)
        p = (dest == (cl + c * _CS)).astype(jnp.bfloat16)
        o_ref[0] += jnp.dot(p, _bf(oc), preferred_element_type=jnp.float32)
        return 0

    jax.lax.fori_loop(0, nch, attn_body, 0)


def _split_heads(W):
    return jnp.transpose(W.reshape(_D, _H, _E), (1, 0, 2))


def _lsh_attn(x, ap, rot):
    Bb, L, _ = x.shape
    nch = L // _CS
    wq, wk, wv = (_split_heads(ap[n]) for n in ('Wq', 'Wk', 'Wv'))
    bq, bk, bv = (ap[n].reshape(_H, 1, _E) for n in ('bq', 'bk', 'bv'))
    R = rot.shape[-1]
    out = pl.pallas_call(
        functools.partial(_lsh_kernel, L=L, nch=nch),
        grid=(Bb, _H),
        in_specs=[
            pl.BlockSpec((1, L, _D), lambda b, h: (b, 0, 0)),
            pl.BlockSpec((1, _D, _E), lambda b, h: (h, 0, 0)),
            pl.BlockSpec((1, _D, _E), lambda b, h: (h, 0, 0)),
            pl.BlockSpec((1, _D, _E), lambda b, h: (h, 0, 0)),
            pl.BlockSpec((1, 1, _E), lambda b, h: (h, 0, 0)),
            pl.BlockSpec((1, 1, _E), lambda b, h: (h, 0, 0)),
            pl.BlockSpec((1, 1, _E), lambda b, h: (h, 0, 0)),
            pl.BlockSpec((_E, R), lambda b, h: (0, 0)),
        ],
        out_specs=pl.BlockSpec((1, L, _E), lambda b, h: (b * _H + h, 0, 0)),
        out_shape=jax.ShapeDtypeStruct((Bb * _H, L, _E), jnp.float32),
        scratch_shapes=[pltpu.VMEM((L, _E), jnp.float32)] * 3,
    )(x, wq, wk, wv, bq, bk, bv, jnp.asarray(rot))
    return out.reshape(Bb, _H, L, _E)


def _cross_kernel(xq_ref, xkv_ref, wq_ref, wk_ref, wv_ref,
                  bq_ref, bk_ref, bv_ref, o_ref):
    xqb = _bf(xq_ref[0])
    xkb = _bf(xkv_ref[0])
    q = (jnp.dot(xqb, _bf(wq_ref[0]), preferred_element_type=jnp.float32)
         + bq_ref[0])
    k = (jnp.dot(xkb, _bf(wk_ref[0]), preferred_element_type=jnp.float32)
         + bk_ref[0])
    v = (jnp.dot(xkb, _bf(wv_ref[0]), preferred_element_type=jnp.float32)
         + bv_ref[0])
    dots = jax.lax.dot_general(
        _bf(q), _bf(k), (((1,), (1,)), ((), ())),
        preferred_element_type=jnp.float32) / 8.0
    mx = jnp.max(dots, axis=1, keepdims=True)
    e = jnp.exp(dots - mx)
    a = _bf(e / jnp.sum(e, axis=1, keepdims=True))
    o_ref[0] = jnp.dot(a, _bf(v), preferred_element_type=jnp.float32)


def _cross_attn(xq, xkv, ap):
    Bb, Lq, _ = xq.shape
    Lk = xkv.shape[1]
    wq, wk, wv = (_split_heads(ap[n]) for n in ('Wq', 'Wk', 'Wv'))
    bq, bk, bv = (ap[n].reshape(_H, 1, _E) for n in ('bq', 'bk', 'bv'))
    out = pl.pallas_call(
        _cross_kernel,
        grid=(Bb, _H),
        in_specs=[
            pl.BlockSpec((1, Lq, _D), lambda b, h: (b, 0, 0)),
            pl.BlockSpec((1, Lk, _D), lambda b, h: (b, 0, 0)),
            pl.BlockSpec((1, _D, _E), lambda b, h: (h, 0, 0)),
            pl.BlockSpec((1, _D, _E), lambda b, h: (h, 0, 0)),
            pl.BlockSpec((1, _D, _E), lambda b, h: (h, 0, 0)),
            pl.BlockSpec((1, 1, _E), lambda b, h: (h, 0, 0)),
            pl.BlockSpec((1, 1, _E), lambda b, h: (h, 0, 0)),
            pl.BlockSpec((1, 1, _E), lambda b, h: (h, 0, 0)),
        ],
        out_specs=pl.BlockSpec((1, Lq, _E), lambda b, h: (b * _H + h, 0, 0)),
        out_shape=jax.ShapeDtypeStruct((Bb * _H, Lq, _E), jnp.float32),
    )(xq, xkv, wq, wk, wv, bq, bk, bv)
    return out.reshape(Bb, _H, Lq, _E)


def _mm_ln_kernel(x_ref, w_ref, b_ref, r_ref, g_ref, be_ref, o_ref):
    y = (r_ref[...] + jnp.dot(_bf(x_ref[...]), _bf(w_ref[...]),
                              preferred_element_type=jnp.float32) + b_ref[...])
    mu = jnp.mean(y, axis=1, keepdims=True)
    var = jnp.mean((y - mu) ** 2, axis=1, keepdims=True)
    o_ref[...] = (y - mu) / jnp.sqrt(var + 1e-5) * g_ref[...] + be_ref[...]


def _mm_ln(xin, W, bvec, resid, g, be, BM=512):
    Bb, L, _ = xin.shape
    M = Bb * L
    K = xin.shape[-1]
    x2 = xin.reshape(M, K)
    r2 = resid.reshape(M, _D)
    out = pl.pallas_call(
        _mm_ln_kernel,
        grid=(M // BM,),
        in_specs=[
            pl.BlockSpec((BM, K), lambda i: (i, 0)),
            pl.BlockSpec((K, _D), lambda i: (0, 0)),
            pl.BlockSpec((1, _D), lambda i: (0, 0)),
            pl.BlockSpec((BM, _D), lambda i: (i, 0)),
            pl.BlockSpec((1, _D), lambda i: (0, 0)),
            pl.BlockSpec((1, _D), lambda i: (0, 0)),
        ],
        out_specs=pl.BlockSpec((BM, _D), lambda i: (i, 0)),
        out_shape=jax.ShapeDtypeStruct((M, _D), jnp.float32),
    )(x2, W, bvec.reshape(1, _D), r2, g.reshape(1, _D), be.reshape(1, _D))
    return out.reshape(Bb, L, _D)


def _ffn1_kernel(x_ref, w_ref, b_ref, o_ref):
    o_ref[...] = jax.nn.gelu(
        jnp.dot(_bf(x_ref[...]), _bf(w_ref[...]),
                preferred_element_type=jnp.float32) + b_ref[...])


def _ffn1(xin, W1, b1, BM=512):
    Bb, L, _ = xin.shape
    M = Bb * L
    x2 = xin.reshape(M, _D)
    out = pl.pallas_call(
        _ffn1_kernel,
        grid=(M // BM,),
        in_specs=[
            pl.BlockSpec((BM, _D), lambda i: (i, 0)),
            pl.BlockSpec((_D, _FF), lambda i: (0, 0)),
            pl.BlockSpec((1, _FF), lambda i: (0, 0)),
        ],
        out_specs=pl.BlockSpec((BM, _FF), lambda i: (i, 0)),
        out_shape=jax.ShapeDtypeStruct((M, _FF), jnp.float32),
    )(x2, W1, b1.reshape(1, _FF))
    return out.reshape(Bb, L, _FF)


def _ln_kernel(x_ref, g_ref, b_ref, o_ref):
    x = x_ref[...]
    mu = jnp.mean(x, axis=1, keepdims=True)
    var = jnp.mean((x - mu) ** 2, axis=1, keepdims=True)
    o_ref[...] = (x - mu) / jnp.sqrt(var + 1e-5) * g_ref[...] + b_ref[...]


def _ln(xin, g, be, BM=512):
    Bb, L, _ = xin.shape
    M = Bb * L
    out = pl.pallas_call(
        _ln_kernel,
        grid=(M // BM,),
        in_specs=[
            pl.BlockSpec((BM, _D), lambda i: (i, 0)),
            pl.BlockSpec((1, _D), lambda i: (0, 0)),
            pl.BlockSpec((1, _D), lambda i: (0, 0)),
        ],
        out_specs=pl.BlockSpec((BM, _D), lambda i: (i, 0)),
        out_shape=jax.ShapeDtypeStruct((M, _D), jnp.float32),
    )(xin.reshape(M, _D), g.reshape(1, _D), be.reshape(1, _D))
    return out.reshape(Bb, L, _D)


def _final_kernel(x_ref, g_ref, b_ref, w_ref, pb_ref, o_ref):
    x = x_ref[0]
    mu = jnp.mean(x, axis=1, keepdims=True)
    var = jnp.mean((x - mu) ** 2, axis=1, keepdims=True)
    xn = (x - mu) / jnp.sqrt(var + 1e-5) * g_ref[...] + b_ref[...]
    o_ref[0] = jnp.dot(xn, w_ref[...],
                       preferred_element_type=jnp.float32) + pb_ref[...]


def _final(xin, dp):
    Bb, L, _ = xin.shape
    C = dp['projW'].shape[-1]
    Wp = jnp.pad(dp['projW'], ((0, 0), (0, 128 - C)))
    bp = jnp.pad(dp['projb'], (0, 128 - C)).reshape(1, 128)
    out = pl.pallas_call(
        _final_kernel,
        grid=(Bb,),
        in_specs=[
            pl.BlockSpec((1, L, _D), lambda b: (b, 0, 0)),
            pl.BlockSpec((1, _D), lambda b: (0, 0)),
            pl.BlockSpec((1, _D), lambda b: (0, 0)),
            pl.BlockSpec((_D, 128), lambda b: (0, 0)),
            pl.BlockSpec((1, 128), lambda b: (0, 0)),
        ],
        out_specs=pl.BlockSpec((1, L, 128), lambda b: (b, 0, 0)),
        out_shape=jax.ShapeDtypeStruct((Bb, L, 128), jnp.float32),
    )(xin, dp['ng'].reshape(1, _D), dp['nb'].reshape(1, _D), Wp, bp)
    return out[:, :, :C]


def _merge_heads(h):
    Bb, H, L, E = h.shape
    return jnp.transpose(h, (0, 2, 1, 3)).reshape(Bb, L, H * E)


def _enc_layer(x, lp, rot):
    heads = _merge_heads(_lsh_attn(x, lp['attn'], rot))
    x1 = _mm_ln(heads, lp['attn']['Wo'], lp['attn']['bo'], x,
                lp['n1g'], lp['n1b'])
    y = _ffn1(x1, lp['W1'], lp['b1'])
    return _mm_ln(y, lp['W2'], lp['b2'], x1, lp['n2g'], lp['n2b'])


def _dec_layer(x, enc, lp, rot):
    heads = _merge_heads(_lsh_attn(x, lp['self'], rot))
    x1 = _mm_ln(heads, lp['self']['Wo'], lp['self']['bo'], x,
                lp['n1g'], lp['n1b'])
    ch = _merge_heads(_cross_attn(x1, enc, lp['cross']))
    x2 = _mm_ln(ch, lp['cross']['Wo'], lp['cross']['bo'], x1,
                lp['n2g'], lp['n2b'])
    y = _ffn1(x2, lp['W1'], lp['b1'])
    return _mm_ln(y, lp['W2'], lp['b2'], x2, lp['n3g'], lp['n3b'])


def kernel(x_enc, x_mark_enc, x_dec, x_mark_dec, params):
    p = params
    x = _embed(x_enc, x_mark_enc, p['enc_emb'], _PE[x_enc.shape[1]])
    for lp in p['encoder']['layers']:
        x = _enc_layer(x, lp, _ROT_ENC)
    enc = _ln(x, p['encoder']['ng'], p['encoder']['nb'])
    d = _embed(x_dec, x_mark_dec, p['dec_emb'], _PE[x_dec.shape[1]])
    for lp in p['decoder']['layers']:
        d = _dec_layer(d, enc, lp, _ROT_DEC)
    out = _final(d, p['decoder'])
    return out[:, -_PRED:, :]


# speedup vs baseline: 1.3363x; 1.3363x over previous
import functools
import numpy as np
import jax
import jax.numpy as jnp
from jax.experimental import pallas as pl
from jax.experimental.pallas import tpu as pltpu

_D = 768
_H = 12
_E = 64
_FF = 3072
_PRED = 512
_CS = 64

_ROT_ENC = np.random.RandomState(42).randn(_E, 16).astype(np.float32)
_ROT_DEC = np.random.RandomState(42).randn(_E, 8).astype(np.float32)


def _pos_emb(L, d):
    pos = np.arange(L, dtype=np.float32)[:, None]
    div = np.exp(np.arange(0, d, 2, dtype=np.float32) * (-np.log(10000.0) / d))
    pe = np.zeros((L, d), dtype=np.float32)
    pe[:, 0::2] = np.sin(pos * div)
    pe[:, 1::2] = np.cos(pos * div)
    return pe

_PE = {2048: _pos_emb(2048, _D), 1024: _pos_emb(1024, _D)}


def _embed_kernel(xcat_ref, w_ref, pe_ref, o_ref):
    o_ref[0] = (jnp.dot(xcat_ref[0], w_ref[...],
                        preferred_element_type=jnp.float32) + pe_ref[...])


def _embed(x, mark, ep, pe):
    Bb, L, C = x.shape
    xcat = jnp.concatenate(
        [jnp.roll(x, 1, axis=1), x, jnp.roll(x, -1, axis=1), mark], axis=-1)
    K = xcat.shape[-1]
    Kp = 32
    xcat = jnp.pad(xcat, ((0, 0), (0, 0), (0, Kp - K)))
    w = jnp.concatenate([ep['conv'][0], ep['conv'][1], ep['conv'][2],
                         ep['timeW']], axis=0)
    w = jnp.pad(w, ((0, Kp - K), (0, 0)))
    return pl.pallas_call(
        _embed_kernel,
        grid=(Bb,),
        in_specs=[
            pl.BlockSpec((1, L, Kp), lambda b: (b, 0, 0)),
            pl.BlockSpec((Kp, _D), lambda b: (0, 0)),
            pl.BlockSpec((L, _D), lambda b: (0, 0)),
        ],
        out_specs=pl.BlockSpec((1, L, _D), lambda b: (b, 0, 0)),
        out_shape=jax.ShapeDtypeStruct((Bb, L, _D), jnp.float32),
    )(xcat, w, jnp.asarray(pe))


def _lsh_kernel(x_ref, wq_ref, wk_ref, wv_ref, bq_ref, bk_ref, bv_ref,
                rot_ref, o_ref, sq_ref, sk_ref, sv_ref, so_ref, *, L, nch):
    x = x_ref[0]
    q = jnp.dot(x, wq_ref[0], preferred_element_type=jnp.float32) + bq_ref[0]
    k = jnp.dot(x, wk_ref[0], preferred_element_type=jnp.float32) + bk_ref[0]
    v = jnp.dot(x, wv_ref[0], preferred_element_type=jnp.float32) + bv_ref[0]

    nb = 2 * rot_ref.shape[-1]
    rotated = jnp.dot(q, rot_ref[...], preferred_element_type=jnp.float32)
    cat = jnp.concatenate([rotated, -rotated], axis=1)
    m = jnp.max(cat, axis=1, keepdims=True)
    lane = jax.lax.broadcasted_iota(jnp.int32, (L, nb), 1)
    bucket = jnp.min(jnp.where(cat == m, lane, nb), axis=1, keepdims=True)

    onehot = (bucket == lane).astype(jnp.float32)
    cum = onehot
    s = 1
    while s < L:
        cum = cum + jnp.concatenate(
            [jnp.zeros((s, nb), jnp.float32), cum[:L - s]], axis=0)
        s *= 2
    hist = cum[L - 1:L, :]
    off = hist
    s = 1
    while s < nb:
        off = off + jnp.concatenate(
            [jnp.zeros((1, s), jnp.float32), off[:, :nb - s]], axis=1)
        s *= 2
    off = off - hist
    rank = jnp.sum(onehot * cum, axis=1, keepdims=True)
    offsel = jnp.sum(onehot * off, axis=1, keepdims=True)
    dest = (offsel + rank - 1.0).astype(jnp.int32)

    pfull = (dest == jax.lax.broadcasted_iota(jnp.int32, (L, L), 1)
             ).astype(jnp.float32)
    dn_g = (((0,), (0,)), ((), ()))
    sq_ref[...] = jax.lax.dot_general(
        pfull, q, dn_g, preferred_element_type=jnp.float32)
    sk_ref[...] = jax.lax.dot_general(
        pfull, k, dn_g, preferred_element_type=jnp.float32)
    sv_ref[...] = jax.lax.dot_general(
        pfull, v, dn_g, preferred_element_type=jnp.float32)

    def attn_body(c, _):
        pc = jnp.where(c == 0, nch - 1, c - 1)
        ic = pl.ds(pl.multiple_of(c * _CS, _CS), _CS)
        ip = pl.ds(pl.multiple_of(pc * _CS, _CS), _CS)
        sq = sq_ref[ic, :]
        k2 = jnp.concatenate([sk_ref[ip, :], sk_ref[ic, :]], axis=0)
        v2 = jnp.concatenate([sv_ref[ip, :], sv_ref[ic, :]], axis=0)
        dots = jax.lax.dot_general(
            sq, k2, (((1,), (1,)), ((), ())),
            preferred_element_type=jnp.float32) / 8.0
        mx = jnp.max(dots, axis=1, keepdims=True)
        e = jnp.exp(dots - mx)
        a = e / jnp.sum(e, axis=1, keepdims=True)
        so_ref[ic, :] = jnp.dot(a, v2, preferred_element_type=jnp.float32)
        return 0

    jax.lax.fori_loop(0, nch, attn_body, 0)

    o_ref[0] = jax.lax.dot_general(
        pfull, so_ref[...], (((1,), (0,)), ((), ())),
        preferred_element_type=jnp.float32)


def _split_heads(W):
    return jnp.transpose(W.reshape(_D, _H, _E), (1, 0, 2))


def _lsh_attn(x, ap, rot):
    Bb, L, _ = x.shape
    nch = L // _CS
    wq, wk, wv = (_split_heads(ap[n]) for n in ('Wq', 'Wk', 'Wv'))
    bq, bk, bv = (ap[n].reshape(_H, 1, _E) for n in ('bq', 'bk', 'bv'))
    R = rot.shape[-1]
    out = pl.pallas_call(
        functools.partial(_lsh_kernel, L=L, nch=nch),
        grid=(Bb, _H),
        in_specs=[
            pl.BlockSpec((1, L, _D), lambda b, h: (b, 0, 0)),
            pl.BlockSpec((1, _D, _E), lambda b, h: (h, 0, 0)),
            pl.BlockSpec((1, _D, _E), lambda b, h: (h, 0, 0)),
            pl.BlockSpec((1, _D, _E), lambda b, h: (h, 0, 0)),
            pl.BlockSpec((1, 1, _E), lambda b, h: (h, 0, 0)),
            pl.BlockSpec((1, 1, _E), lambda b, h: (h, 0, 0)),
            pl.BlockSpec((1, 1, _E), lambda b, h: (h, 0, 0)),
            pl.BlockSpec((_E, R), lambda b, h: (0, 0)),
        ],
        out_specs=pl.BlockSpec((1, L, _E), lambda b, h: (b * _H + h, 0, 0)),
        out_shape=jax.ShapeDtypeStruct((Bb * _H, L, _E), jnp.float32),
        scratch_shapes=[pltpu.VMEM((L, _E), jnp.float32)] * 4,
    )(x, wq, wk, wv, bq, bk, bv, jnp.asarray(rot))
    return out.reshape(Bb, _H, L, _E)


def _cross_kernel(xq_ref, xkv_ref, wq_ref, wk_ref, wv_ref,
                  bq_ref, bk_ref, bv_ref, o_ref):
    q = (jnp.dot(xq_ref[0], wq_ref[0], preferred_element_type=jnp.float32)
         + bq_ref[0])
    k = (jnp.dot(xkv_ref[0], wk_ref[0], preferred_element_type=jnp.float32)
         + bk_ref[0])
    v = (jnp.dot(xkv_ref[0], wv_ref[0], preferred_element_type=jnp.float32)
         + bv_ref[0])
    dots = jax.lax.dot_general(
        q, k, (((1,), (1,)), ((), ())),
        preferred_element_type=jnp.float32) / 8.0
    mx = jnp.max(dots, axis=1, keepdims=True)
    e = jnp.exp(dots - mx)
    a = e / jnp.sum(e, axis=1, keepdims=True)
    o_ref[0] = jnp.dot(a, v, preferred_element_type=jnp.float32)


def _cross_attn(xq, xkv, ap):
    Bb, Lq, _ = xq.shape
    Lk = xkv.shape[1]
    wq, wk, wv = (_split_heads(ap[n]) for n in ('Wq', 'Wk', 'Wv'))
    bq, bk, bv = (ap[n].reshape(_H, 1, _E) for n in ('bq', 'bk', 'bv'))
    out = pl.pallas_call(
        _cross_kernel,
        grid=(Bb, _H),
        in_specs=[
            pl.BlockSpec((1, Lq, _D), lambda b, h: (b, 0, 0)),
            pl.BlockSpec((1, Lk, _D), lambda b, h: (b, 0, 0)),
            pl.BlockSpec((1, _D, _E), lambda b, h: (h, 0, 0)),
            pl.BlockSpec((1, _D, _E), lambda b, h: (h, 0, 0)),
            pl.BlockSpec((1, _D, _E), lambda b, h: (h, 0, 0)),
            pl.BlockSpec((1, 1, _E), lambda b, h: (h, 0, 0)),
            pl.BlockSpec((1, 1, _E), lambda b, h: (h, 0, 0)),
            pl.BlockSpec((1, 1, _E), lambda b, h: (h, 0, 0)),
        ],
        out_specs=pl.BlockSpec((1, Lq, _E), lambda b, h: (b * _H + h, 0, 0)),
        out_shape=jax.ShapeDtypeStruct((Bb * _H, Lq, _E), jnp.float32),
    )(xq, xkv, wq, wk, wv, bq, bk, bv)
    return out.reshape(Bb, _H, Lq, _E)


def _mm_ln_kernel(x_ref, w_ref, b_ref, r_ref, g_ref, be_ref, o_ref):
    y = (r_ref[...] + jnp.dot(x_ref[...], w_ref[...],
                              preferred_element_type=jnp.float32) + b_ref[...])
    mu = jnp.mean(y, axis=1, keepdims=True)
    var = jnp.mean((y - mu) ** 2, axis=1, keepdims=True)
    o_ref[...] = (y - mu) / jnp.sqrt(var + 1e-5) * g_ref[...] + be_ref[...]


def _mm_ln(xin, W, bvec, resid, g, be, BM=512):
    Bb, L, _ = xin.shape
    M = Bb * L
    K = xin.shape[-1]
    x2 = xin.reshape(M, K)
    r2 = resid.reshape(M, _D)
    out = pl.pallas_call(
        _mm_ln_kernel,
        grid=(M // BM,),
        in_specs=[
            pl.BlockSpec((BM, K), lambda i: (i, 0)),
            pl.BlockSpec((K, _D), lambda i: (0, 0)),
            pl.BlockSpec((1, _D), lambda i: (0, 0)),
            pl.BlockSpec((BM, _D), lambda i: (i, 0)),
            pl.BlockSpec((1, _D), lambda i: (0, 0)),
            pl.BlockSpec((1, _D), lambda i: (0, 0)),
        ],
        out_specs=pl.BlockSpec((BM, _D), lambda i: (i, 0)),
        out_shape=jax.ShapeDtypeStruct((M, _D), jnp.float32),
    )(x2, W, bvec.reshape(1, _D), r2, g.reshape(1, _D), be.reshape(1, _D))
    return out.reshape(Bb, L, _D)


def _ffn1_kernel(x_ref, w_ref, b_ref, o_ref):
    o_ref[...] = jax.nn.gelu(
        jnp.dot(x_ref[...], w_ref[...],
                preferred_element_type=jnp.float32) + b_ref[...])


def _ffn1(xin, W1, b1, BM=512):
    Bb, L, _ = xin.shape
    M = Bb * L
    x2 = xin.reshape(M, _D)
    out = pl.pallas_call(
        _ffn1_kernel,
        grid=(M // BM,),
        in_specs=[
            pl.BlockSpec((BM, _D), lambda i: (i, 0)),
            pl.BlockSpec((_D, _FF), lambda i: (0, 0)),
            pl.BlockSpec((1, _FF), lambda i: (0, 0)),
        ],
        out_specs=pl.BlockSpec((BM, _FF), lambda i: (i, 0)),
        out_shape=jax.ShapeDtypeStruct((M, _FF), jnp.float32),
    )(x2, W1, b1.reshape(1, _FF))
    return out.reshape(Bb, L, _FF)


def _ln_kernel(x_ref, g_ref, b_ref, o_ref):
    x = x_ref[...]
    mu = jnp.mean(x, axis=1, keepdims=True)
    var = jnp.mean((x - mu) ** 2, axis=1, keepdims=True)
    o_ref[...] = (x - mu) / jnp.sqrt(var + 1e-5) * g_ref[...] + b_ref[...]


def _ln(xin, g, be, BM=512):
    Bb, L, _ = xin.shape
    M = Bb * L
    out = pl.pallas_call(
        _ln_kernel,
        grid=(M // BM,),
        in_specs=[
            pl.BlockSpec((BM, _D), lambda i: (i, 0)),
            pl.BlockSpec((1, _D), lambda i: (0, 0)),
            pl.BlockSpec((1, _D), lambda i: (0, 0)),
        ],
        out_specs=pl.BlockSpec((BM, _D), lambda i: (i, 0)),
        out_shape=jax.ShapeDtypeStruct((M, _D), jnp.float32),
    )(xin.reshape(M, _D), g.reshape(1, _D), be.reshape(1, _D))
    return out.reshape(Bb, L, _D)


def _final_kernel(x_ref, g_ref, b_ref, w_ref, pb_ref, o_ref):
    x = x_ref[0]
    mu = jnp.mean(x, axis=1, keepdims=True)
    var = jnp.mean((x - mu) ** 2, axis=1, keepdims=True)
    xn = (x - mu) / jnp.sqrt(var + 1e-5) * g_ref[...] + b_ref[...]
    o_ref[0] = jnp.dot(xn, w_ref[...],
                       preferred_element_type=jnp.float32) + pb_ref[...]


def _final(xin, dp):
    Bb, L, _ = xin.shape
    C = dp['projW'].shape[-1]
    Wp = jnp.pad(dp['projW'], ((0, 0), (0, 128 - C)))
    bp = jnp.pad(dp['projb'], (0, 128 - C)).reshape(1, 128)
    out = pl.pallas_call(
        _final_kernel,
        grid=(Bb,),
        in_specs=[
            pl.BlockSpec((1, L, _D), lambda b: (b, 0, 0)),
            pl.BlockSpec((1, _D), lambda b: (0, 0)),
            pl.BlockSpec((1, _D), lambda b: (0, 0)),
            pl.BlockSpec((_D, 128), lambda b: (0, 0)),
            pl.BlockSpec((1, 128), lambda b: (0, 0)),
        ],
        out_specs=pl.BlockSpec((1, L, 128), lambda b: (b, 0, 0)),
        out_shape=jax.ShapeDtypeStruct((Bb, L, 128), jnp.float32),
    )(xin, dp['ng'].reshape(1, _D), dp['nb'].reshape(1, _D), Wp, bp)
    return out[:, :, :C]


def _merge_heads(h):
    Bb, H, L, E = h.shape
    return jnp.transpose(h, (0, 2, 1, 3)).reshape(Bb, L, H * E)


def _enc_layer(x, lp, rot):
    heads = _merge_heads(_lsh_attn(x, lp['attn'], rot))
    x1 = _mm_ln(heads, lp['attn']['Wo'], lp['attn']['bo'], x,
                lp['n1g'], lp['n1b'])
    y = _ffn1(x1, lp['W1'], lp['b1'])
    return _mm_ln(y, lp['W2'], lp['b2'], x1, lp['n2g'], lp['n2b'])


def _dec_layer(x, enc, lp, rot):
    heads = _merge_heads(_lsh_attn(x, lp['self'], rot))
    x1 = _mm_ln(heads, lp['self']['Wo'], lp['self']['bo'], x,
                lp['n1g'], lp['n1b'])
    ch = _merge_heads(_cross_attn(x1, enc, lp['cross']))
    x2 = _mm_ln(ch, lp['cross']['Wo'], lp['cross']['bo'], x1,
                lp['n2g'], lp['n2b'])
    y = _ffn1(x2, lp['W1'], lp['b1'])
    return _mm_ln(y, lp['W2'], lp['b2'], x2, lp['n3g'], lp['n3b'])


def kernel(x_enc, x_mark_enc, x_dec, x_mark_dec, params):
    p = params
    x = _embed(x_enc, x_mark_enc, p['enc_emb'], _PE[x_enc.shape[1]])
    for lp in p['encoder']['layers']:
        x = _enc_layer(x, lp, _ROT_ENC)
    enc = _ln(x, p['encoder']['ng'], p['encoder']['nb'])
    d = _embed(x_dec, x_mark_dec, p['dec_emb'], _PE[x_dec.shape[1]])
    for lp in p['decoder']['layers']:
        d = _dec_layer(d, enc, lp, _ROT_DEC)
    out = _final(d, p['decoder'])
    return out[:, -_PRED:, :]


# speedup vs baseline: 1.5443x; 1.1557x over previous
import functools
import numpy as np
import jax
import jax.numpy as jnp
from jax import lax
from jax.experimental import pallas as pl
from jax.experimental.pallas import tpu as pltpu
from jax.experimental.pallas import tpu_sc as plsc

_D = 768
_H = 12
_E = 64
_FF = 3072
_PRED = 512
_CS = 64

_ROT_ENC = np.random.RandomState(42).randn(_E, 16).astype(np.float32)
_ROT_DEC = np.random.RandomState(42).randn(_E, 8).astype(np.float32)


def _pos_emb(L, d):
    pos = np.arange(L, dtype=np.float32)[:, None]
    div = np.exp(np.arange(0, d, 2, dtype=np.float32) * (-np.log(10000.0) / d))
    pe = np.zeros((L, d), dtype=np.float32)
    pe[:, 0::2] = np.sin(pos * div)
    pe[:, 1::2] = np.cos(pos * div)
    return pe

_PE = {2048: _pos_emb(2048, _D), 1024: _pos_emb(1024, _D)}


def _embed_kernel(xcat_ref, w_ref, pe_ref, o_ref):
    o_ref[0] = (jnp.dot(xcat_ref[0], w_ref[...],
                        preferred_element_type=jnp.float32) + pe_ref[...])


def _embed(x, mark, ep, pe):
    Bb, L, C = x.shape
    xcat = jnp.concatenate(
        [jnp.roll(x, 1, axis=1), x, jnp.roll(x, -1, axis=1), mark], axis=-1)
    K = xcat.shape[-1]
    Kp = 32
    xcat = jnp.pad(xcat, ((0, 0), (0, 0), (0, Kp - K)))
    w = jnp.concatenate([ep['conv'][0], ep['conv'][1], ep['conv'][2],
                         ep['timeW']], axis=0)
    w = jnp.pad(w, ((0, Kp - K), (0, 0)))
    return pl.pallas_call(
        _embed_kernel,
        grid=(Bb,),
        in_specs=[
            pl.BlockSpec((1, L, Kp), lambda b: (b, 0, 0)),
            pl.BlockSpec((Kp, _D), lambda b: (0, 0)),
            pl.BlockSpec((L, _D), lambda b: (0, 0)),
        ],
        out_specs=pl.BlockSpec((1, L, _D), lambda b: (b, 0, 0)),
        out_shape=jax.ShapeDtypeStruct((Bb, L, _D), jnp.float32),
    )(xcat, w, jnp.asarray(pe))


def _bf(x):
    return x.astype(jnp.bfloat16)


def _lsh_prep_kernel(x_ref, wc_ref, bc_ref, rot_ref, qkv_ref, gd_ref,
                     *, L, nch):
    x = x_ref[0]
    qkv = (jnp.dot(x, wc_ref[0], preferred_element_type=jnp.float32)
           + bc_ref[0])
    q = qkv[:, 0:_E]

    nb = 2 * rot_ref.shape[-1]
    rotated = jnp.dot(q, rot_ref[...], preferred_element_type=jnp.float32)
    cat = jnp.concatenate([rotated, -rotated], axis=1)
    m = jnp.max(cat, axis=1, keepdims=True)
    lane = jax.lax.broadcasted_iota(jnp.int32, (L, nb), 1)
    bucket = jnp.min(jnp.where(cat == m, lane, nb), axis=1, keepdims=True)

    onehot = (bucket == lane).astype(jnp.float32)
    cum = onehot
    s = 1
    while s < L:
        cum = cum + jnp.concatenate(
            [jnp.zeros((s, nb), jnp.float32), cum[:L - s]], axis=0)
        s *= 2
    hist = cum[L - 1:L, :]
    off = hist
    s = 1
    while s < nb:
        off = off + jnp.concatenate(
            [jnp.zeros((1, s), jnp.float32), off[:, :nb - s]], axis=1)
        s *= 2
    off = off - hist
    rank = jnp.sum(onehot * cum, axis=1, keepdims=True)
    offsel = jnp.sum(onehot * off, axis=1, keepdims=True)
    dest = (offsel + rank - 1.0).astype(jnp.int32)

    bh = pl.program_id(0) * _H + pl.program_id(1)
    qkv_ref[0] = jnp.concatenate(
        [qkv, jnp.zeros((L, _E), jnp.float32)], axis=1)
    gd_ref[0] = jnp.transpose(dest) + bh * L


def _sorted_attn_kernel(sqkv_ref, o_ref, *, L, nch):
    def attn_body(c, _):
        pc = jnp.where(c == 0, nch - 1, c - 1)
        ic = pl.ds(pl.multiple_of(c * _CS, _CS), _CS)
        ip = pl.ds(pl.multiple_of(pc * _CS, _CS), _CS)
        cur = sqkv_ref[0, ic, :]
        prv = sqkv_ref[0, ip, :]
        sq = cur[:, 0:_E]
        k2 = jnp.concatenate([prv[:, _E:2 * _E], cur[:, _E:2 * _E]], axis=0)
        v2 = jnp.concatenate([prv[:, 2 * _E:3 * _E], cur[:, 2 * _E:3 * _E]],
                             axis=0)
        dots = jax.lax.dot_general(
            sq, k2, (((1,), (1,)), ((), ())),
            preferred_element_type=jnp.float32) / 8.0
        mx = jnp.max(dots, axis=1, keepdims=True)
        e = jnp.exp(dots - mx)
        a = e / jnp.sum(e, axis=1, keepdims=True)
        oc = jnp.dot(a, v2, preferred_element_type=jnp.float32)
        o_ref[0, ic, :] = jnp.concatenate(
            [oc, jnp.zeros((_CS, _E), jnp.float32)], axis=1)
        return 0

    jax.lax.fori_loop(0, nch, attn_body, 0)


_NW = 32


def _sc_permute(table, gidx, gather):
    R, D = table.shape
    k = gidx.shape[1]
    rows_pw = k * 128
    mesh = plsc.VectorSubcoreMesh(core_axis_name="c", subcore_axis_name="s")

    @functools.partial(
        pl.kernel, mesh=mesh,
        out_type=jax.ShapeDtypeStruct((R, D), jnp.float32),
        scratch_types=[
            pltpu.VMEM((k, 128), jnp.int32),
            pltpu.VMEM((128, D), jnp.float32),
            pltpu.VMEM((128, D), jnp.float32),
            pltpu.SemaphoreType.DMA,
            pltpu.SemaphoreType.DMA,
        ],
    )
    def go(table_hbm, gidx_hbm, out_hbm, idx_v, rows_a, rows_b, rsem, wsem):
        wid = lax.axis_index("s") * 2 + lax.axis_index("c")
        pltpu.sync_copy(gidx_hbm.at[wid], idx_v)
        base = wid * rows_pw
        bufs = (rows_a, rows_b)

        def rd(j):
            if gather:
                return pltpu.async_copy(table_hbm.at[idx_v.at[j]],
                                        bufs[j % 2], rsem)
            return pltpu.async_copy(table_hbm.at[pl.ds(base + j * 128, 128)],
                                    bufs[j % 2], rsem)

        def wr(j):
            if gather:
                return pltpu.async_copy(bufs[j % 2],
                                        out_hbm.at[pl.ds(base + j * 128, 128)],
                                        wsem)
            return pltpu.async_copy(bufs[j % 2], out_hbm.at[idx_v.at[j]],
                                    wsem)

        reads = {0: rd(0)}
        writes = {}
        for j in range(k):
            reads.pop(j).wait()
            writes[j] = wr(j)
            if j + 1 < k:
                if j >= 1:
                    writes.pop(j - 1).wait()
                reads[j + 1] = rd(j + 1)
        for j in sorted(writes):
            writes.pop(j).wait()

    return go(table, gidx)


def _split_heads(W):
    return jnp.transpose(W.reshape(_D, _H, _E), (1, 0, 2))


def _qkv_cat(ap, names):
    w = jnp.concatenate([_split_heads(ap[n]) for n in names], axis=2)
    b = jnp.concatenate([ap[n].reshape(_H, 1, _E) for n in
                         ('bq', 'bk', 'bv')], axis=2)
    return w, b


def _lsh_attn(x, ap, rot):
    Bb, L, _ = x.shape
    nch = L // _CS
    wc, bc = _qkv_cat(ap, ('Wq', 'Wk', 'Wv'))
    R = rot.shape[-1]
    BH = Bb * _H
    qkv, gd = pl.pallas_call(
        functools.partial(_lsh_prep_kernel, L=L, nch=nch),
        grid=(Bb, _H),
        in_specs=[
            pl.BlockSpec((1, L, _D), lambda b, h: (b, 0, 0)),
            pl.BlockSpec((1, _D, 3 * _E), lambda b, h: (h, 0, 0)),
            pl.BlockSpec((1, 1, 3 * _E), lambda b, h: (h, 0, 0)),
            pl.BlockSpec((_E, R), lambda b, h: (0, 0)),
        ],
        out_specs=[
            pl.BlockSpec((1, L, 4 * _E), lambda b, h: (b * _H + h, 0, 0)),
            pl.BlockSpec((1, 1, L), lambda b, h: (b * _H + h, 0, 0)),
        ],
        out_shape=[
            jax.ShapeDtypeStruct((BH, L, 4 * _E), jnp.float32),
            jax.ShapeDtypeStruct((BH, 1, L), jnp.int32),
        ],
    )(x, wc, bc, jnp.asarray(rot))

    gidx = gd.reshape(_NW, (BH * L) // (_NW * 128), 128)
    sqkv = _sc_permute(qkv.reshape(BH * L, 4 * _E), gidx, gather=False)
    so = pl.pallas_call(
        functools.partial(_sorted_attn_kernel, L=L, nch=nch),
        grid=(BH,),
        in_specs=[pl.BlockSpec((1, L, 4 * _E), lambda i: (i, 0, 0))],
        out_specs=pl.BlockSpec((1, L, 2 * _E), lambda i: (i, 0, 0)),
        out_shape=jax.ShapeDtypeStruct((BH, L, 2 * _E), jnp.float32),
    )(sqkv.reshape(BH, L, 4 * _E))
    out = _sc_permute(so.reshape(BH * L, 2 * _E), gidx, gather=True)
    return out.reshape(Bb, _H, L, 2 * _E)[:, :, :, :_E]


def _cross_kernel(xq_ref, xkv_ref, wq_ref, wkv_ref, bc_ref, o_ref):
    q = (jnp.dot(xq_ref[0], wq_ref[0], preferred_element_type=jnp.float32)
         + bc_ref[0][:, 0:_E])
    kv = (jnp.dot(xkv_ref[0], wkv_ref[0], preferred_element_type=jnp.float32)
          + bc_ref[0][:, _E:3 * _E])
    dots = jax.lax.dot_general(
        q, kv[:, 0:_E], (((1,), (1,)), ((), ())),
        preferred_element_type=jnp.float32) / 8.0
    mx = jnp.max(dots, axis=1, keepdims=True)
    e = jnp.exp(dots - mx)
    a = e / jnp.sum(e, axis=1, keepdims=True)
    o_ref[0] = jnp.dot(a, kv[:, _E:2 * _E],
                       preferred_element_type=jnp.float32)


def _cross_attn(xq, xkv, ap):
    Bb, Lq, _ = xq.shape
    Lk = xkv.shape[1]
    wc, bc = _qkv_cat(ap, ('Wq', 'Wk', 'Wv'))
    wq = wc[:, :, 0:_E]
    wkv = wc[:, :, _E:3 * _E]
    out = pl.pallas_call(
        _cross_kernel,
        grid=(Bb, _H),
        in_specs=[
            pl.BlockSpec((1, Lq, _D), lambda b, h: (b, 0, 0)),
            pl.BlockSpec((1, Lk, _D), lambda b, h: (b, 0, 0)),
            pl.BlockSpec((1, _D, _E), lambda b, h: (h, 0, 0)),
            pl.BlockSpec((1, _D, 2 * _E), lambda b, h: (h, 0, 0)),
            pl.BlockSpec((1, 1, 3 * _E), lambda b, h: (h, 0, 0)),
        ],
        out_specs=pl.BlockSpec((1, Lq, _E), lambda b, h: (b * _H + h, 0, 0)),
        out_shape=jax.ShapeDtypeStruct((Bb * _H, Lq, _E), jnp.float32),
    )(xq, xkv, wq, wkv, bc)
    return out.reshape(Bb, _H, Lq, _E)


def _mm_ln_kernel(x_ref, w_ref, b_ref, r_ref, g_ref, be_ref, o_ref):
    y = (r_ref[...] + jnp.dot(_bf(x_ref[...]), _bf(w_ref[...]),
                              preferred_element_type=jnp.float32) + b_ref[...])
    mu = jnp.mean(y, axis=1, keepdims=True)
    var = jnp.mean((y - mu) ** 2, axis=1, keepdims=True)
    o_ref[...] = (y - mu) / jnp.sqrt(var + 1e-5) * g_ref[...] + be_ref[...]


def _mm_ln(xin, W, bvec, resid, g, be, BM=512):
    Bb, L, _ = xin.shape
    M = Bb * L
    K = xin.shape[-1]
    x2 = xin.reshape(M, K)
    r2 = resid.reshape(M, _D)
    out = pl.pallas_call(
        _mm_ln_kernel,
        grid=(M // BM,),
        in_specs=[
            pl.BlockSpec((BM, K), lambda i: (i, 0)),
            pl.BlockSpec((K, _D), lambda i: (0, 0)),
            pl.BlockSpec((1, _D), lambda i: (0, 0)),
            pl.BlockSpec((BM, _D), lambda i: (i, 0)),
            pl.BlockSpec((1, _D), lambda i: (0, 0)),
            pl.BlockSpec((1, _D), lambda i: (0, 0)),
        ],
        out_specs=pl.BlockSpec((BM, _D), lambda i: (i, 0)),
        out_shape=jax.ShapeDtypeStruct((M, _D), jnp.float32),
    )(x2, W, bvec.reshape(1, _D), r2, g.reshape(1, _D), be.reshape(1, _D))
    return out.reshape(Bb, L, _D)


def _ffn1_kernel(x_ref, w_ref, b_ref, o_ref):
    o_ref[...] = jax.nn.gelu(
        jnp.dot(_bf(x_ref[...]), _bf(w_ref[...]),
                preferred_element_type=jnp.float32) + b_ref[...])


def _ffn1(xin, W1, b1, BM=512):
    Bb, L, _ = xin.shape
    M = Bb * L
    x2 = xin.reshape(M, _D)
    out = pl.pallas_call(
        _ffn1_kernel,
        grid=(M // BM,),
        in_specs=[
            pl.BlockSpec((BM, _D), lambda i: (i, 0)),
            pl.BlockSpec((_D, _FF), lambda i: (0, 0)),
            pl.BlockSpec((1, _FF), lambda i: (0, 0)),
        ],
        out_specs=pl.BlockSpec((BM, _FF), lambda i: (i, 0)),
        out_shape=jax.ShapeDtypeStruct((M, _FF), jnp.float32),
    )(x2, W1, b1.reshape(1, _FF))
    return out.reshape(Bb, L, _FF)


def _ln_kernel(x_ref, g_ref, b_ref, o_ref):
    x = x_ref[...]
    mu = jnp.mean(x, axis=1, keepdims=True)
    var = jnp.mean((x - mu) ** 2, axis=1, keepdims=True)
    o_ref[...] = (x - mu) / jnp.sqrt(var + 1e-5) * g_ref[...] + b_ref[...]


def _ln(xin, g, be, BM=512):
    Bb, L, _ = xin.shape
    M = Bb * L
    out = pl.pallas_call(
        _ln_kernel,
        grid=(M // BM,),
        in_specs=[
            pl.BlockSpec((BM, _D), lambda i: (i, 0)),
            pl.BlockSpec((1, _D), lambda i: (0, 0)),
            pl.BlockSpec((1, _D), lambda i: (0, 0)),
        ],
        out_specs=pl.BlockSpec((BM, _D), lambda i: (i, 0)),
        out_shape=jax.ShapeDtypeStruct((M, _D), jnp.float32),
    )(xin.reshape(M, _D), g.reshape(1, _D), be.reshape(1, _D))
    return out.reshape(Bb, L, _D)


def _final_kernel(x_ref, g_ref, b_ref, w_ref, pb_ref, o_ref):
    x = x_ref[0]
    mu = jnp.mean(x, axis=1, keepdims=True)
    var = jnp.mean((x - mu) ** 2, axis=1, keepdims=True)
    xn = (x - mu) / jnp.sqrt(var + 1e-5) * g_ref[...] + b_ref[...]
    o_ref[0] = jnp.dot(xn, w_ref[...],
                       preferred_element_type=jnp.float32) + pb_ref[...]


def _final(xin, dp):
    Bb, L, _ = xin.shape
    C = dp['projW'].shape[-1]
    Wp = jnp.pad(dp['projW'], ((0, 0), (0, 128 - C)))
    bp = jnp.pad(dp['projb'], (0, 128 - C)).reshape(1, 128)
    out = pl.pallas_call(
        _final_kernel,
        grid=(Bb,),
        in_specs=[
            pl.BlockSpec((1, L, _D), lambda b: (b, 0, 0)),
            pl.BlockSpec((1, _D), lambda b: (0, 0)),
            pl.BlockSpec((1, _D), lambda b: (0, 0)),
            pl.BlockSpec((_D, 128), lambda b: (0, 0)),
            pl.BlockSpec((1, 128), lambda b: (0, 0)),
        ],
        out_specs=pl.BlockSpec((1, L, 128), lambda b: (b, 0, 0)),
        out_shape=jax.ShapeDtypeStruct((Bb, L, 128), jnp.float32),
    )(xin, dp['ng'].reshape(1, _D), dp['nb'].reshape(1, _D), Wp, bp)
    return out[:, :, :C]


def _merge_heads(h):
    Bb, H, L, E = h.shape
    return jnp.transpose(h, (0, 2, 1, 3)).reshape(Bb, L, H * E)


def _enc_layer(x, lp, rot):
    heads = _merge_heads(_lsh_attn(x, lp['attn'], rot))
    x1 = _mm_ln(heads, lp['attn']['Wo'], lp['attn']['bo'], x,
                lp['n1g'], lp['n1b'])
    y = _ffn1(x1, lp['W1'], lp['b1'])
    return _mm_ln(y, lp['W2'], lp['b2'], x1, lp['n2g'], lp['n2b'])


def _dec_layer(x, enc, lp, rot):
    heads = _merge_heads(_lsh_attn(x, lp['self'], rot))
    x1 = _mm_ln(heads, lp['self']['Wo'], lp['self']['bo'], x,
                lp['n1g'], lp['n1b'])
    ch = _merge_heads(_cross_attn(x1, enc, lp['cross']))
    x2 = _mm_ln(ch, lp['cross']['Wo'], lp['cross']['bo'], x1,
                lp['n2g'], lp['n2b'])
    y = _ffn1(x2, lp['W1'], lp['b1'])
    return _mm_ln(y, lp['W2'], lp['b2'], x2, lp['n3g'], lp['n3b'])


def kernel(x_enc, x_mark_enc, x_dec, x_mark_dec, params):
    p = params
    x = _embed(x_enc, x_mark_enc, p['enc_emb'], _PE[x_enc.shape[1]])
    for lp in p['encoder']['layers']:
        x = _enc_layer(x, lp, _ROT_ENC)
    enc = _ln(x, p['encoder']['ng'], p['encoder']['nb'])
    d = _embed(x_dec, x_mark_dec, p['dec_emb'], _PE[x_dec.shape[1]])
    for lp in p['decoder']['layers']:
        d = _dec_layer(d, enc, lp, _ROT_DEC)
    out = _final(d, p['decoder'])
    return out[:, -_PRED:, :]
